# baseline (speedup 1.0000x reference)
import functools

import jax
import jax.numpy as jnp
from jax import lax
from jax.experimental import pallas as pl
from jax.experimental.pallas import tpu as pltpu
from jax.experimental.pallas import tpu_sc as plsc

NFEAT = 56
DIM = 128
CBITS = 7
NCHUNK = 8
CROWS = 1 << CBITS
TROWS = NCHUNK * CROWS
NLANE = 16
NCORE = 2
NSUBCORE = 16
XBLK = 4096


def _table_body(tab_ref, t_ref):
    r = lax.broadcasted_iota(jnp.int32, (TROWS, NFEAT), 0)
    i = lax.broadcasted_iota(jnp.int32, (TROWS, NFEAT), 1)
    c = r // CROWS
    b = r % CROWS
    j = i - c * CBITS
    inch = (j >= 0) & (j < CBITS)
    bit = jnp.right_shift(b, jnp.clip(j, 0, CBITS - 1)) & 1
    tab = tab_ref[...]
    m1 = (inch & (bit == 1)).astype(jnp.float32)
    m0 = (inch & (bit == 0)).astype(jnp.float32)
    dn = (((1,), (0,)), ((), ()))
    t_ref[...] = (
        lax.dot_general(m1, tab[:, 1, :], dn, preferred_element_type=jnp.float32)
        + lax.dot_general(m0, tab[:, 0, :], dn, preferred_element_type=jnp.float32))


def _pack_body(x_ref, w_ref):
    i = lax.broadcasted_iota(jnp.int32, (NFEAT, NCHUNK), 0)
    c = lax.broadcasted_iota(jnp.int32, (NFEAT, NCHUNK), 1)
    p = jnp.where((i // CBITS) == c,
                  jnp.left_shift(1, i % CBITS), 0).astype(jnp.float32)
    xf = x_ref[...].astype(jnp.float32)
    wf = lax.dot_general(p, xf, (((0,), (0,)), ((), ())),
                         preferred_element_type=jnp.float32)
    cc = lax.broadcasted_iota(jnp.int32, (NCHUNK, XBLK), 0)
    w_ref[...] = wf.astype(jnp.int32) + cc * CROWS


@functools.lru_cache(maxsize=None)
def _make_sc(n, npad):
    nw = NCORE * NSUBCORE
    rpw = npad // nw
    assert npad % nw == 0 and rpw % NLANE == 0
    bn = 128
    assert rpw % bn == 0
    npair = bn // 2
    ngrp = bn // NLANE

    def body(codes_hbm, t_hbm, out_hbm, w2_v, w_v, t_v, o_vs, sems):
        wid = lax.axis_index("s") * NCORE + lax.axis_index("c")
        row0 = wid * rpw
        nvalid = jnp.minimum(rpw, n - row0)
        nblk_w = (nvalid + bn - 1) // bn
        cps = [pltpu.async_copy(t_hbm, t_v, sems[0])]
        for c in range(NCHUNK):
            cps.append(pltpu.async_copy(
                codes_hbm.at[pl.ds(c * npad + row0, rpw)],
                w2_v.at[pl.ds(c * rpw, rpw)], sems[0]))
        for cp in cps:
            cp.wait()
        iota = lax.iota(jnp.int32, NLANE)
        iota_w = iota * NCHUNK

        hi_mask = jnp.int32(-65536)

        def unpack2(v):
            a = lax.bitcast_convert_type(lax.shift_left(v, 16), jnp.float32)
            b = lax.bitcast_convert_type(v & hi_mask, jnp.float32)
            return a, b

        def accum_row(o_v, wvec, lane0, r):
            rows = []
            for ci in range(NCHUNK):
                oc = wvec[lane0 + ci] * (DIM // 2)
                rows.append([plsc.bitcast(t_v[pl.ds(oc + g * NLANE, NLANE)],
                                          jnp.bfloat16) for g in range(4)])
            while len(rows) > 1:
                rows = [[a + b for a, b in zip(r1, r2)]
                        for r1, r2 in zip(rows[::2], rows[1::2])]
            for g in range(4):
                a, b = unpack2(plsc.bitcast(rows[0][g], jnp.int32))
                o_v[r, pl.ds(g * 32, NLANE)] = a
                o_v[r, pl.ds(g * 32 + NLANE, NLANE)] = b

        def do_block(blk, o_v, sem):
            r0loc = jnp.minimum(blk * bn, nvalid - bn)
            r0 = row0 + r0loc

            @pl.when(blk >= 2)
            def _wait_prev():
                pltpu.make_async_copy(
                    o_v, out_hbm.at[pl.ds(0, bn)], sem).wait()

            def xpose_group(g, carry):
                wbase = iota_w + g * (NLANE * NCHUNK)
                for c in range(NCHUNK):
                    cvec = w2_v[pl.ds(c * rpw + r0loc + g * NLANE, NLANE)]
                    plsc.store_scatter(w_v, [wbase + c], cvec)
                return carry

            lax.fori_loop(0, ngrp, xpose_group, 0)

            def pair(p2, carry2):
                wvec = w_v[pl.ds(p2 * NLANE, NLANE)] & jnp.int32(TROWS - 1)
                accum_row(o_v, wvec, 0, 2 * p2)
                accum_row(o_v, wvec, NCHUNK, 2 * p2 + 1)
                return carry2

            lax.fori_loop(0, npair, pair, 0)
            pltpu.async_copy(o_v, out_hbm.at[pl.ds(r0, bn)], sem)

        def body2(k, carry):
            for phase in range(2):
                blk = 2 * k + phase

                @pl.when(blk < nblk_w)
                def _run():
                    do_block(blk, o_vs[phase], sems[phase])

            return carry

        lax.fori_loop(0, (rpw // bn + 1) // 2, body2, 0)
        for phase in range(2):
            pltpu.make_async_copy(
                o_vs[phase], out_hbm.at[pl.ds(0, bn)], sems[phase]).wait()

    return pl.kernel(
        body,
        out_type=jax.ShapeDtypeStruct((n, DIM), jnp.float32),
        mesh=plsc.VectorSubcoreMesh(core_axis_name="c", subcore_axis_name="s",
                                    num_cores=NCORE, num_subcores=NSUBCORE),
        compiler_params=pltpu.CompilerParams(needs_layout_passes=False,
                                             use_tc_tiling_on_sc=True),
        scratch_types=[
            pltpu.VMEM((NCHUNK * (npad // nw),), jnp.int32),
            pltpu.VMEM((bn * NCHUNK,), jnp.int32),
            pltpu.VMEM((TROWS * DIM // 2,), jnp.int32),
            [pltpu.VMEM((bn, DIM), jnp.float32) for _ in range(2)],
            [pltpu.SemaphoreType.DMA for _ in range(2)],
        ],
    )


def kernel(x, tables):
    n = x.shape[0]
    npad = ((n + XBLK - 1) // XBLK) * XBLK
    t = pl.pallas_call(
        _table_body,
        out_shape=jax.ShapeDtypeStruct((TROWS, DIM), jnp.float32),
    )(tables)
    tb = lax.bitcast_convert_type(
        t.reshape(TROWS, 4, 2, NLANE).astype(jnp.bfloat16),
        jnp.uint16).astype(jnp.uint32)
    t_pk = lax.bitcast_convert_type(
        tb[:, :, 0, :] | (tb[:, :, 1, :] << 16), jnp.int32)
    codes = pl.pallas_call(
        _pack_body,
        grid=(npad // XBLK,),
        in_specs=[pl.BlockSpec((NFEAT, XBLK), lambda i: (0, i))],
        out_specs=pl.BlockSpec((NCHUNK, XBLK), lambda i: (0, i)),
        out_shape=jax.ShapeDtypeStruct((NCHUNK, npad), jnp.int32),
    )(x.T)
    return _make_sc(n, npad)(codes.reshape(-1), t_pk.reshape(-1))


# speedup vs baseline: 1.0323x; 1.0323x over previous
import functools

import jax
import jax.numpy as jnp
from jax import lax
from jax.experimental import pallas as pl
from jax.experimental.pallas import tpu as pltpu
from jax.experimental.pallas import tpu_sc as plsc

NFEAT = 56
DIM = 128
CBITS = 7
NCHUNK = 8
CROWS = 1 << CBITS
TROWS = NCHUNK * CROWS
NLANE = 16
NCORE = 2
NSUBCORE = 16
XBLK = 4096


def _table_body(tab_ref, t_ref):
    r = lax.broadcasted_iota(jnp.int32, (TROWS, NFEAT), 0)
    i = lax.broadcasted_iota(jnp.int32, (TROWS, NFEAT), 1)
    c = r // CROWS
    b = r % CROWS
    j = i - c * CBITS
    inch = (j >= 0) & (j < CBITS)
    bit = jnp.right_shift(b, jnp.clip(j, 0, CBITS - 1)) & 1
    tab = tab_ref[...]
    m1 = (inch & (bit == 1)).astype(jnp.float32)
    m0 = (inch & (bit == 0)).astype(jnp.float32)
    dn = (((1,), (0,)), ((), ()))
    t_ref[...] = (
        lax.dot_general(m1, tab[:, 1, :], dn, preferred_element_type=jnp.float32)
        + lax.dot_general(m0, tab[:, 0, :], dn, preferred_element_type=jnp.float32))


def _pack_body(x_ref, w_ref):
    i = lax.broadcasted_iota(jnp.int32, (NFEAT, NCHUNK), 0)
    c = lax.broadcasted_iota(jnp.int32, (NFEAT, NCHUNK), 1)
    p = jnp.where((i // CBITS) == c,
                  jnp.left_shift(1, i % CBITS), 0).astype(jnp.float32)
    xf = x_ref[...].astype(jnp.float32)
    wf = lax.dot_general(p, xf, (((0,), (0,)), ((), ())),
                         preferred_element_type=jnp.float32)
    cc = lax.broadcasted_iota(jnp.int32, (NCHUNK, XBLK), 0)
    w_ref[...] = wf.astype(jnp.int32) + cc * CROWS


@functools.lru_cache(maxsize=None)
def _make_sc(n, npad):
    nw = NCORE * NSUBCORE
    rpw = npad // nw
    assert npad % nw == 0 and rpw % NLANE == 0
    bn = 128
    assert rpw % bn == 0
    npair = bn // 2
    ngrp = bn // NLANE

    def body(codes_hbm, t_hbm, out_hbm, w2_v, w_v, t_v, o_vs, sems):
        wid = lax.axis_index("s") * NCORE + lax.axis_index("c")
        row0 = wid * rpw
        nvalid = jnp.minimum(rpw, n - row0)
        nblk_w = (nvalid + bn - 1) // bn
        cps = [pltpu.async_copy(t_hbm, t_v, sems[0])]
        for c in range(NCHUNK):
            cps.append(pltpu.async_copy(
                codes_hbm.at[pl.ds(c * npad + row0, rpw)],
                w2_v.at[pl.ds(c * rpw, rpw)], sems[0]))
        for cp in cps:
            cp.wait()
        iota = lax.iota(jnp.int32, NLANE)
        iota_w = iota * NCHUNK

        hi_mask = jnp.int32(-65536)

        def unpack2(v):
            a = lax.bitcast_convert_type(lax.shift_left(v, 16), jnp.float32)
            b = lax.bitcast_convert_type(v & hi_mask, jnp.float32)
            return a, b

        def accum_row(o_v, wvec, lane0, r):
            chains = [None, None]
            for ci in range(NCHUNK):
                oc = wvec[lane0 + ci] * (DIM // 2)
                row = [plsc.bitcast(t_v[pl.ds(oc + g * NLANE, NLANE)],
                                    jnp.bfloat16) for g in range(4)]
                k = ci & 1
                chains[k] = row if chains[k] is None else [
                    a + b for a, b in zip(chains[k], row)]
            total = [a + b for a, b in zip(chains[0], chains[1])]
            for g in range(4):
                a, b = unpack2(plsc.bitcast(total[g], jnp.int32))
                o_v[r, pl.ds(g * 32, NLANE)] = a
                o_v[r, pl.ds(g * 32 + NLANE, NLANE)] = b

        def do_block(blk, o_v, sem):
            r0loc = jnp.minimum(blk * bn, nvalid - bn)
            r0 = row0 + r0loc

            @pl.when(blk >= 2)
            def _wait_prev():
                pltpu.make_async_copy(
                    o_v, out_hbm.at[pl.ds(0, bn)], sem).wait()

            def xpose_group(g, carry):
                wbase = iota_w + g * (NLANE * NCHUNK)
                for c in range(NCHUNK):
                    cvec = w2_v[pl.ds(c * rpw + r0loc + g * NLANE, NLANE)]
                    plsc.store_scatter(w_v, [wbase + c], cvec)
                return carry

            lax.fori_loop(0, ngrp, xpose_group, 0)

            def pair(p2, carry2):
                wvec = w_v[pl.ds(p2 * NLANE, NLANE)] & jnp.int32(TROWS - 1)
                accum_row(o_v, wvec, 0, 2 * p2)
                accum_row(o_v, wvec, NCHUNK, 2 * p2 + 1)
                return carry2

            lax.fori_loop(0, npair, pair, 0)
            pltpu.async_copy(o_v, out_hbm.at[pl.ds(r0, bn)], sem)

        def body2(k, carry):
            for phase in range(2):
                blk = 2 * k + phase

                @pl.when(blk < nblk_w)
                def _run():
                    do_block(blk, o_vs[phase], sems[phase])

            return carry

        lax.fori_loop(0, (rpw // bn + 1) // 2, body2, 0)
        for phase in range(2):
            pltpu.make_async_copy(
                o_vs[phase], out_hbm.at[pl.ds(0, bn)], sems[phase]).wait()

    return pl.kernel(
        body,
        out_type=jax.ShapeDtypeStruct((n, DIM), jnp.float32),
        mesh=plsc.VectorSubcoreMesh(core_axis_name="c", subcore_axis_name="s",
                                    num_cores=NCORE, num_subcores=NSUBCORE),
        compiler_params=pltpu.CompilerParams(needs_layout_passes=False,
                                             use_tc_tiling_on_sc=True),
        scratch_types=[
            pltpu.VMEM((NCHUNK * (npad // nw),), jnp.int32),
            pltpu.VMEM((bn * NCHUNK,), jnp.int32),
            pltpu.VMEM((TROWS * DIM // 2,), jnp.int32),
            [pltpu.VMEM((bn, DIM), jnp.float32) for _ in range(2)],
            [pltpu.SemaphoreType.DMA for _ in range(2)],
        ],
    )


def kernel(x, tables):
    n = x.shape[0]
    npad = ((n + XBLK - 1) // XBLK) * XBLK
    t = pl.pallas_call(
        _table_body,
        out_shape=jax.ShapeDtypeStruct((TROWS, DIM), jnp.float32),
    )(tables)
    tb = lax.bitcast_convert_type(
        t.reshape(TROWS, 4, 2, NLANE).astype(jnp.bfloat16),
        jnp.uint16).astype(jnp.uint32)
    t_pk = lax.bitcast_convert_type(
        tb[:, :, 0, :] | (tb[:, :, 1, :] << 16), jnp.int32)
    codes = pl.pallas_call(
        _pack_body,
        grid=(npad // XBLK,),
        in_specs=[pl.BlockSpec((NFEAT, XBLK), lambda i: (0, i))],
        out_specs=pl.BlockSpec((NCHUNK, XBLK), lambda i: (0, i)),
        out_shape=jax.ShapeDtypeStruct((NCHUNK, npad), jnp.int32),
    )(x.T)
    return _make_sc(n, npad)(codes.reshape(-1), t_pk.reshape(-1))
